# baseline (speedup 1.0000x reference)
import jax
import jax.numpy as jnp
import numpy as np
from jax import lax
from jax.experimental import pallas as pl
from jax.experimental.pallas import tpu as pltpu

K = 8
B = 128
V = 32768
ROWS_PER_BLK = 16
NBLK = B // ROWS_PER_BLK
CW = 512
NC = V // CW

_ROTS = ((13, 15, 26, 6), (17, 29, 16, 24))


def _np_threefry2x32(k0, k1, x0, x1):
    with np.errstate(over='ignore'):
        ks = (k0, k1, k0 ^ k1 ^ np.uint32(0x1BD11BDA))
        x0 = x0 + ks[0]
        x1 = x1 + ks[1]
        for i in range(5):
            for d in _ROTS[i % 2]:
                x0 = x0 + x1
                x1 = (x1 << np.uint32(d)) | (x1 >> np.uint32(32 - d))
                x1 = x0 ^ x1
            x0 = x0 + ks[(i + 1) % 3]
            x1 = x1 + ks[(i + 2) % 3] + np.uint32(i + 1)
    return x0, x1


_KD = np.array(
    [_np_threefry2x32(np.uint32(0), np.uint32(42), np.uint32(0), np.uint32(k))
     for k in range(K)],
    dtype=np.uint32).astype(np.int64).astype(np.int32)


def _threefry_bits(k0, k1, x1):
    ks2 = k0 ^ k1 ^ np.int32(0x1BD11BDA)
    ks = (k0, k1, ks2)
    x0 = jnp.full_like(x1, k0)
    for i in range(5):
        for d in _ROTS[i % 2]:
            x0 = x0 + x1
            x1 = lax.shift_left(x1, np.int32(d)) | lax.shift_right_logical(
                x1, np.int32(32 - d))
            x1 = x0 ^ x1
        x0 = x0 + ks[(i + 1) % 3]
        x1 = x1 + (ks[(i + 2) % 3] + np.int32(i + 1))
    return x0 ^ x1


def _body(kd_ref, logits_ref, out_ref):
    k = pl.program_id(0)
    b = pl.program_id(1)
    k0 = kd_ref[k, 0]
    k1 = kd_ref[k, 1]

    row = lax.broadcasted_iota(jnp.int32, (ROWS_PER_BLK, CW), 0)
    col = lax.broadcasted_iota(jnp.int32, (ROWS_PER_BLK, CW), 1)
    base = (b * ROWS_PER_BLK + row) * V + col + k1

    tiny = np.float32(np.finfo(np.float32).tiny)
    span = np.float32(1.0) - tiny

    def chunk(j, carry):
        vm, ci = carry
        x1 = base + j * CW
        bits = _threefry_bits(k0, k1, x1)
        mant = lax.shift_right_logical(bits, np.int32(9)) | np.int32(0x3F800000)
        floats = lax.bitcast_convert_type(mant, jnp.float32) - np.float32(1.0)
        u = jnp.maximum(tiny, floats * span + tiny)
        g = -jnp.log(-jnp.log(u))
        v = logits_ref[0, :, pl.ds(j * CW, CW)] + g
        take = v > vm
        vm = jnp.where(take, v, vm)
        ci = jnp.where(take, j, ci)
        return vm, ci

    vm0 = jnp.full((ROWS_PER_BLK, CW), -jnp.inf, dtype=jnp.float32)
    ci0 = jnp.zeros((ROWS_PER_BLK, CW), dtype=jnp.int32)
    vm, ci = lax.fori_loop(0, NC, chunk, (vm0, ci0), unroll=64)

    m = jnp.max(vm, axis=-1, keepdims=True)
    gidx = ci * CW + col
    cand = jnp.where(vm == m, gidx, V)
    out_ref[0, 0, :] = jnp.min(cand, axis=-1).astype(jnp.int32)


@jax.jit
def kernel(logits):
    kd = jnp.asarray(_KD)

    out = pl.pallas_call(
        _body,
        grid=(K, NBLK),
        in_specs=[
            pl.BlockSpec(memory_space=pltpu.SMEM),
            pl.BlockSpec((1, ROWS_PER_BLK, V), lambda k, b: (k, b, 0)),
        ],
        out_specs=pl.BlockSpec((1, 1, ROWS_PER_BLK),
                               lambda k, b: (k * NBLK + b, 0, 0)),
        out_shape=jax.ShapeDtypeStruct((K * NBLK, 1, ROWS_PER_BLK), jnp.int32),
        compiler_params=pltpu.CompilerParams(
            dimension_semantics=("parallel", "parallel")),
    )(kd, logits)

    return out.reshape(K, B).T.reshape(B, 1, K)


# speedup vs baseline: 1.0169x; 1.0169x over previous
import jax
import jax.numpy as jnp
import numpy as np
from jax import lax
from jax.experimental import pallas as pl
from jax.experimental.pallas import tpu as pltpu

K = 8
B = 128
V = 32768
ROWS_PER_BLK = 32
NBLK = B // ROWS_PER_BLK
CW = 512
NC = V // CW

_ROTS = ((13, 15, 26, 6), (17, 29, 16, 24))


def _np_threefry2x32(k0, k1, x0, x1):
    with np.errstate(over='ignore'):
        ks = (k0, k1, k0 ^ k1 ^ np.uint32(0x1BD11BDA))
        x0 = x0 + ks[0]
        x1 = x1 + ks[1]
        for i in range(5):
            for d in _ROTS[i % 2]:
                x0 = x0 + x1
                x1 = (x1 << np.uint32(d)) | (x1 >> np.uint32(32 - d))
                x1 = x0 ^ x1
            x0 = x0 + ks[(i + 1) % 3]
            x1 = x1 + ks[(i + 2) % 3] + np.uint32(i + 1)
    return x0, x1


_KD = np.array(
    [_np_threefry2x32(np.uint32(0), np.uint32(42), np.uint32(0), np.uint32(k))
     for k in range(K)],
    dtype=np.uint32).astype(np.int64).astype(np.int32)


def _threefry_bits(k0, k1, x1):
    ks2 = k0 ^ k1 ^ np.int32(0x1BD11BDA)
    ks = (k0, k1, ks2)
    x0 = jnp.full_like(x1, k0)
    for i in range(5):
        for d in _ROTS[i % 2]:
            x0 = x0 + x1
            x1 = lax.shift_left(x1, np.int32(d)) | lax.shift_right_logical(
                x1, np.int32(32 - d))
            x1 = x0 ^ x1
        x0 = x0 + ks[(i + 1) % 3]
        x1 = x1 + (ks[(i + 2) % 3] + np.int32(i + 1))
    return x0 ^ x1


def _body(kd_ref, logits_ref, out_ref):
    k = pl.program_id(0)
    b = pl.program_id(1)
    k0 = kd_ref[k, 0]
    k1 = kd_ref[k, 1]

    row = lax.broadcasted_iota(jnp.int32, (ROWS_PER_BLK, CW), 0)
    col = lax.broadcasted_iota(jnp.int32, (ROWS_PER_BLK, CW), 1)
    base = (b * ROWS_PER_BLK + row) * V + col + k1

    tiny = np.float32(np.finfo(np.float32).tiny)

    def chunk(j, carry):
        vm, ci = carry
        x1 = base + j * CW
        bits = _threefry_bits(k0, k1, x1)
        mant = lax.shift_right_logical(bits, np.int32(9)) | np.int32(0x3F800000)
        floats = lax.bitcast_convert_type(mant, jnp.float32) - np.float32(1.0)
        u = floats + tiny
        g = -jnp.log(-jnp.log(u))
        v = logits_ref[0, :, pl.ds(j * CW, CW)] + g
        take = v > vm
        vm = jnp.where(take, v, vm)
        ci = jnp.where(take, j, ci)
        return vm, ci

    vm0 = jnp.full((ROWS_PER_BLK, CW), -jnp.inf, dtype=jnp.float32)
    ci0 = jnp.zeros((ROWS_PER_BLK, CW), dtype=jnp.int32)
    vm, ci = lax.fori_loop(0, NC, chunk, (vm0, ci0), unroll=64)

    m = jnp.max(vm, axis=-1, keepdims=True)
    gidx = ci * CW + col
    cand = jnp.where(vm == m, gidx, V)
    out_ref[0, 0, :] = jnp.min(cand, axis=-1).astype(jnp.int32)


@jax.jit
def kernel(logits):
    kd = jnp.asarray(_KD)

    out = pl.pallas_call(
        _body,
        grid=(K, NBLK),
        in_specs=[
            pl.BlockSpec(memory_space=pltpu.SMEM),
            pl.BlockSpec((1, ROWS_PER_BLK, V), lambda k, b: (k, b, 0)),
        ],
        out_specs=pl.BlockSpec((1, 1, ROWS_PER_BLK),
                               lambda k, b: (k * NBLK + b, 0, 0)),
        out_shape=jax.ShapeDtypeStruct((K * NBLK, 1, ROWS_PER_BLK), jnp.int32),
        compiler_params=pltpu.CompilerParams(
            dimension_semantics=("parallel", "parallel")),
    )(kd, logits)

    return out.reshape(K, B).T.reshape(B, 1, K)


# speedup vs baseline: 1.0288x; 1.0117x over previous
import jax
import jax.numpy as jnp
import numpy as np
from jax import lax
from jax.experimental import pallas as pl
from jax.experimental.pallas import tpu as pltpu

K = 8
B = 128
V = 32768
ROWS_PER_BLK = 32
NBLK = B // ROWS_PER_BLK
CW = 512
NC = V // CW

_ROTS = ((13, 15, 26, 6), (17, 29, 16, 24))


def _np_threefry2x32(k0, k1, x0, x1):
    with np.errstate(over='ignore'):
        ks = (k0, k1, k0 ^ k1 ^ np.uint32(0x1BD11BDA))
        x0 = x0 + ks[0]
        x1 = x1 + ks[1]
        for i in range(5):
            for d in _ROTS[i % 2]:
                x0 = x0 + x1
                x1 = (x1 << np.uint32(d)) | (x1 >> np.uint32(32 - d))
                x1 = x0 ^ x1
            x0 = x0 + ks[(i + 1) % 3]
            x1 = x1 + ks[(i + 2) % 3] + np.uint32(i + 1)
    return x0, x1


_KD = np.array(
    [_np_threefry2x32(np.uint32(0), np.uint32(42), np.uint32(0), np.uint32(k))
     for k in range(K)],
    dtype=np.uint32).astype(np.int64).astype(np.int32)


def _threefry_bits(k0, k1, x1):
    ks2 = k0 ^ k1 ^ np.int32(0x1BD11BDA)
    ks = (k0, k1, ks2)
    x0 = jnp.full_like(x1, k0)
    for i in range(5):
        for d in _ROTS[i % 2]:
            x0 = x0 + x1
            x1 = lax.shift_left(x1, np.int32(d)) | lax.shift_right_logical(
                x1, np.int32(32 - d))
            x1 = x0 ^ x1
        x0 = x0 + ks[(i + 1) % 3]
        x1 = x1 + (ks[(i + 2) % 3] + np.int32(i + 1))
    return x0 ^ x1


def _body(kd_ref, logits_ref, out_ref):
    k = pl.program_id(0)
    b = pl.program_id(1)
    k0 = kd_ref[k, 0]
    k1 = kd_ref[k, 1]

    row = lax.broadcasted_iota(jnp.int32, (ROWS_PER_BLK, CW), 0)
    col = lax.broadcasted_iota(jnp.int32, (ROWS_PER_BLK, CW), 1)
    base = (b * ROWS_PER_BLK + row) * V + col + k1

    tiny = np.float32(np.finfo(np.float32).tiny)

    def chunk(j, carry):
        vm, ci = carry
        x1 = base + j * CW
        bits = _threefry_bits(k0, k1, x1)
        mant = lax.shift_right_logical(bits, np.int32(9)) | np.int32(0x3F800000)
        floats = lax.bitcast_convert_type(mant, jnp.float32) - np.float32(1.0)
        u = floats + tiny
        g = -jnp.log(-jnp.log(u))
        v = logits_ref[0, :, pl.ds(j * CW, CW)] + g
        vmn = jnp.maximum(vm, v)
        ci = jnp.where(vmn > vm, j, ci)
        return vmn, ci

    vm0 = jnp.full((ROWS_PER_BLK, CW), -jnp.inf, dtype=jnp.float32)
    ci0 = jnp.zeros((ROWS_PER_BLK, CW), dtype=jnp.int32)
    vm, ci = lax.fori_loop(0, NC, chunk, (vm0, ci0), unroll=64)

    m = jnp.max(vm, axis=-1, keepdims=True)
    gidx = ci * CW + col
    cand = jnp.where(vm == m, gidx, V)
    out_ref[0, 0, :] = jnp.min(cand, axis=-1).astype(jnp.int32)


@jax.jit
def kernel(logits):
    kd = jnp.asarray(_KD)

    out = pl.pallas_call(
        _body,
        grid=(K, NBLK),
        in_specs=[
            pl.BlockSpec(memory_space=pltpu.SMEM),
            pl.BlockSpec((1, ROWS_PER_BLK, V), lambda k, b: (k, b, 0)),
        ],
        out_specs=pl.BlockSpec((1, 1, ROWS_PER_BLK),
                               lambda k, b: (k * NBLK + b, 0, 0)),
        out_shape=jax.ShapeDtypeStruct((K * NBLK, 1, ROWS_PER_BLK), jnp.int32),
        compiler_params=pltpu.CompilerParams(
            dimension_semantics=("parallel", "parallel")),
    )(kd, logits)

    return out.reshape(K, B).T.reshape(B, 1, K)
